# baseline (speedup 1.0000x reference)
import functools

import jax
import jax.numpy as jnp
from jax.experimental import pallas as pl
from jax.experimental.pallas import tpu as pltpu

EMA_EPS = 1e-12


def _ema_block_kernel(pt_ref, z_ref, out_ref, w_ref, cw_ref, state_ref, *, T):
    k = pl.program_id(0)
    d = pl.program_id(1)
    B, _, DT = z_ref.shape

    @pl.when(d == 0)
    def _():
        p = pt_ref[:, 0, :]
        logd = jnp.log(jnp.maximum(1.0 - p, EMA_EPS))

        jj = jax.lax.broadcasted_iota(jnp.int32, (T, T), 0)
        ii = jax.lax.broadcasted_iota(jnp.int32, (T, T), 1)
        cum_mat = jnp.where(jj <= ii, 1.0, 0.0)
        S = jnp.dot(logd, cum_mat, preferred_element_type=jnp.float32)

        delta = S[:, :, None] - S[:, None, :]
        delta = jnp.where((jj >= ii)[None], delta, -jnp.inf)
        w_ref[...] = jnp.exp(delta) * p[:, None, :]
        cw_ref[...] = jnp.exp(S)

    @pl.when(jnp.logical_and(k == 0, d == 0))
    def _():
        state_ref[...] = jnp.zeros_like(state_ref[...])

    acc = jax.lax.dot_general(
        w_ref[...], z_ref[...],
        dimension_numbers=(((2,), (1,)), ((0,), (0,))),
        preferred_element_type=jnp.float32,
    )

    state = state_ref[:, d, :]
    res = acc + cw_ref[...][:, :, None] * state[:, None, :]
    out_ref[...] = res
    state_ref[:, d, :] = res[:, T - 1, :]


@jax.jit
def kernel(z, pt):
    B, L, D = z.shape
    T = 256
    DT = 256
    K = L // T
    ND = D // DT

    body = functools.partial(_ema_block_kernel, T=T)
    return pl.pallas_call(
        body,
        grid=(K, ND),
        in_specs=[
            pl.BlockSpec((B, 1, T), lambda k, d: (0, 0, k)),
            pl.BlockSpec((B, T, DT), lambda k, d: (0, k, d)),
        ],
        out_specs=pl.BlockSpec((B, T, DT), lambda k, d: (0, k, d)),
        out_shape=jax.ShapeDtypeStruct((B, L, D), jnp.float32),
        scratch_shapes=[
            pltpu.VMEM((B, T, T), jnp.float32),
            pltpu.VMEM((B, T), jnp.float32),
            pltpu.VMEM((B, ND, DT), jnp.float32),
        ],
    )(pt.reshape(B, 1, L), z)


# speedup vs baseline: 1.0617x; 1.0617x over previous
import functools

import jax
import jax.numpy as jnp
from jax.experimental import pallas as pl
from jax.experimental.pallas import tpu as pltpu

EMA_EPS = 1e-12


def _ema_block_kernel(pt_ref, ptn_ref, z_ref, out_ref, w_ref, cw_ref,
                      state_ref, *, T, K):
    k = pl.program_id(0)
    B, _, D = z_ref.shape

    jj = jax.lax.broadcasted_iota(jnp.int32, (T, T), 0)
    ii = jax.lax.broadcasted_iota(jnp.int32, (T, T), 1)
    cum_mat = jnp.where(jj <= ii, 1.0, 0.0)
    tril = (jj >= ii)[None]

    def build(p, slot):
        logd = jnp.log(jnp.maximum(1.0 - p, EMA_EPS))
        S = jnp.dot(logd, cum_mat, preferred_element_type=jnp.float32)
        delta = S[:, :, None] - S[:, None, :]
        delta = jnp.where(tril, delta, -jnp.inf)
        w_ref[slot] = jnp.exp(delta) * p[:, None, :]
        cw_ref[slot] = jnp.exp(S)

    @pl.when(k == 0)
    def _():
        build(pt_ref[:, 0, :], 0)
        state_ref[...] = jnp.zeros_like(state_ref[...])

    slot = jax.lax.rem(k, 2)
    acc = jax.lax.dot_general(
        w_ref[slot], z_ref[...],
        dimension_numbers=(((2,), (1,)), ((0,), (0,))),
        preferred_element_type=jnp.float32,
    )

    @pl.when(k + 1 < K)
    def _():
        build(ptn_ref[:, 0, :], 1 - slot)

    state = state_ref[...]
    res = acc + cw_ref[slot][:, :, None] * state[:, None, :]
    out_ref[...] = res
    state_ref[...] = res[:, T - 1, :]


@jax.jit
def kernel(z, pt):
    B, L, D = z.shape
    T = 256
    K = L // T

    body = functools.partial(_ema_block_kernel, T=T, K=K)
    pt3 = pt.reshape(B, 1, L)
    return pl.pallas_call(
        body,
        grid=(K,),
        in_specs=[
            pl.BlockSpec((B, 1, T), lambda k: (0, 0, k)),
            pl.BlockSpec((B, 1, T), lambda k: (0, 0, jnp.minimum(k + 1, K - 1))),
            pl.BlockSpec((B, T, D), lambda k: (0, k, 0)),
        ],
        out_specs=pl.BlockSpec((B, T, D), lambda k: (0, k, 0)),
        out_shape=jax.ShapeDtypeStruct((B, L, D), jnp.float32),
        scratch_shapes=[
            pltpu.VMEM((2, B, T, T), jnp.float32),
            pltpu.VMEM((2, B, T), jnp.float32),
            pltpu.VMEM((B, D), jnp.float32),
        ],
    )(pt3, pt3, z)


# speedup vs baseline: 1.1151x; 1.0503x over previous
import functools

import jax
import jax.numpy as jnp
from jax.experimental import pallas as pl
from jax.experimental.pallas import tpu as pltpu

EMA_EPS = 1e-12


def _ema_block_kernel(pt_ref, z_ref, out_ref, state_ref, *, T):
    k = pl.program_id(0)
    B, _, D = z_ref.shape

    p = pt_ref[:, 0, :]
    logd = jnp.log(jnp.maximum(1.0 - p, EMA_EPS))

    jj = jax.lax.broadcasted_iota(jnp.int32, (T, T), 0)
    ii = jax.lax.broadcasted_iota(jnp.int32, (T, T), 1)
    cum_mat = jnp.where(jj <= ii, 1.0, 0.0)
    S = jnp.dot(logd, cum_mat, preferred_element_type=jnp.float32)

    delta = S[:, :, None] - S[:, None, :]
    delta = jnp.where((jj >= ii)[None], delta, -jnp.inf)
    W = jnp.exp(delta) * p[:, None, :]

    acc = jax.lax.dot_general(
        W, z_ref[...],
        dimension_numbers=(((2,), (1,)), ((0,), (0,))),
        preferred_element_type=jnp.float32,
    )

    @pl.when(k == 0)
    def _():
        state_ref[...] = jnp.zeros((B, D), jnp.float32)

    state = state_ref[...]
    res = acc + jnp.exp(S)[:, :, None] * state[:, None, :]
    out_ref[...] = res
    state_ref[...] = res[:, T - 1, :]


@jax.jit
def kernel(z, pt):
    B, L, D = z.shape
    T = 256
    K = L // T

    body = functools.partial(_ema_block_kernel, T=T)
    return pl.pallas_call(
        body,
        grid=(K,),
        in_specs=[
            pl.BlockSpec((B, 1, T), lambda k: (0, 0, k)),
            pl.BlockSpec((B, T, D), lambda k: (0, k, 0)),
        ],
        out_specs=pl.BlockSpec((B, T, D), lambda k: (0, k, 0)),
        out_shape=jax.ShapeDtypeStruct((B, L, D), jnp.float32),
        scratch_shapes=[pltpu.VMEM((B, D), jnp.float32)],
    )(pt.reshape(B, 1, L), z)
